# baseline (speedup 1.0000x reference)
import functools

import jax
import jax.numpy as jnp
from jax import lax
from jax.experimental import pallas as pl
from jax.experimental.pallas import tpu as pltpu
from jax.experimental.pallas import tpu_sc as plsc

B = 16384
EMB = 64
L = 16

_info = plsc.get_sparse_core_info()
_NC, _NS = _info.num_cores, _info.num_subcores
NW = _NC * _NS
BPW = B // NW
CH = 128
NCH = BPW // CH

_mesh = plsc.VectorSubcoreMesh(core_axis_name="c", subcore_axis_name="s")


@functools.partial(
    pl.kernel,
    mesh=_mesh,
    compiler_params=pltpu.CompilerParams(
        needs_layout_passes=False, use_tc_tiling_on_sc=False),
    out_type=jax.ShapeDtypeStruct((B,), jnp.float32),
    scratch_types=[
        pltpu.VMEM((NCH, CH), jnp.int32),
        pltpu.VMEM((NCH, CH), jnp.int32),
        pltpu.VMEM((NCH, CH), jnp.int32),
        pltpu.VMEM((BPW, EMB), jnp.float32),
        pltpu.VMEM((BPW, EMB), jnp.float32),
        pltpu.VMEM((BPW, EMB), jnp.float32),
        pltpu.VMEM((BPW,), jnp.float32),
        pltpu.SemaphoreType.DMA,
        pltpu.SemaphoreType.DMA,
        pltpu.SemaphoreType.DMA,
    ],
)
def _lp_kernel(s_hbm, p_hbm, o_hbm, node_hbm, rel_hbm, out_hbm,
               sidx_v, pidx_v, oidx_v, srow_v, prow_v, orow_v, out_v,
               sem_s, sem_p, sem_o):
    wid = lax.axis_index("s") * _NC + lax.axis_index("c")
    base = wid * NCH

    pltpu.sync_copy(s_hbm.at[pl.ds(base, NCH)], sidx_v)
    pltpu.sync_copy(p_hbm.at[pl.ds(base, NCH)], pidx_v)
    pltpu.sync_copy(o_hbm.at[pl.ds(base, NCH)], oidx_v)

    copies = []
    for j in range(NCH):
        rows = pl.ds(j * CH, CH)
        copies.append(pltpu.async_copy(
            node_hbm.at[sidx_v.at[j]], srow_v.at[rows], sem_s))
        copies.append(pltpu.async_copy(
            rel_hbm.at[pidx_v.at[j]], prow_v.at[rows], sem_p))
        copies.append(pltpu.async_copy(
            node_hbm.at[oidx_v.at[j]], orow_v.at[rows], sem_o))
    for c in copies:
        c.wait()

    lane = lax.iota(jnp.int32, 16)
    last_lane = lane == (L - 1)

    def triple_body(i, carry):
        acc = jnp.zeros((L,), jnp.float32)
        for k in range(EMB // L):
            sl = pl.ds(k * L, L)
            acc = acc + srow_v[i, sl] * prow_v[i, sl] * orow_v[i, sl]
        tot = plsc.cumsum(acc)
        plsc.store_scatter(out_v, [jnp.full((L,), i, jnp.int32)], tot,
                           mask=last_lane)
        return carry

    lax.fori_loop(0, BPW, triple_body, 0)

    def sig_body(b, carry):
        sl = pl.ds(b * L, L)
        v = out_v[sl]
        out_v[sl] = 1.0 / (1.0 + jnp.exp(-v))
        return carry

    lax.fori_loop(0, BPW // L, sig_body, 0)

    pltpu.sync_copy(out_v, out_hbm.at[pl.ds(wid * BPW, BPW)])


def kernel(s_idx, p_idx, o_idx, node_emb, rel_emb):
    s2 = s_idx.reshape(B // CH, CH)
    p2 = p_idx.reshape(B // CH, CH)
    o2 = o_idx.reshape(B // CH, CH)
    return _lp_kernel(s2, p2, o2, node_emb, rel_emb)
